# baseline (speedup 1.0000x reference)
import functools
import jax, jax.numpy as jnp
from jax import lax
from jax.experimental import pallas as pl
from jax.experimental.pallas import tpu as pltpu
from jax.experimental.pallas import tpu_sc as plsc

K = 8192
DIM = 32
NPIX = 6272
KCH = 1024

NC = 2
NS = 16
NW = NC * NS
BPW = 208
NPAD = NW * BPW
KPW = K // NW


def _conv(x, w, b, s, p):
    y = jax.lax.conv_general_dilated(x, w, (s, s), [(p, p), (p, p)], dimension_numbers=('NCHW', 'OIHW', 'NCHW'))
    return y + b[None, :, None, None]


def _convt(x, w, b):
    w2 = jnp.flip(w, (2, 3)).transpose(1, 0, 2, 3)
    y = jax.lax.conv_general_dilated(x, w2, (1, 1), [(2, 2), (2, 2)], lhs_dilation=(2, 2), dimension_numbers=('NCHW', 'OIHW', 'NCHW'))
    return y + b[None, :, None, None]


def _res(x, w1, b1, w2, b2):
    t = jax.nn.relu(x)
    t = _conv(t, w1, b1, 1, 1)
    t = jax.nn.relu(t)
    t = _conv(t, w2, b2, 1, 0)
    return x + t


def _vq_body(esq_ref, emb2_ref, zb_ref, zsq_ref, idx_ref, best_ref, bidx_ref):
    i = pl.program_id(0)
    cross2 = jax.lax.dot_general(
        emb2_ref[...], zb_ref[...], (((1,), (0,)), ((), ())),
        preferred_element_type=jnp.float32)
    a = esq_ref[...] + cross2
    d = a + zsq_ref[...]
    cmin = jnp.min(d, axis=0, keepdims=True)
    rows = jax.lax.broadcasted_iota(jnp.int32, d.shape, 0)
    loc = jnp.min(jnp.where(d == cmin, rows, jnp.int32(2 ** 30)), axis=0, keepdims=True)
    gidx = loc + i * KCH

    @pl.when(i == 0)
    def _():
        best_ref[...] = cmin
        bidx_ref[...] = gidx

    @pl.when(i > 0)
    def _():
        upd = cmin < best_ref[...]
        bidx_ref[...] = jnp.where(upd, gidx, bidx_ref[...])
        best_ref[...] = jnp.where(upd, cmin, best_ref[...])

    @pl.when(i == pl.num_programs(0) - 1)
    def _():
        idx_ref[...] = bidx_ref[...]


def _vq_argmin(ze, codebook):
    zb = ze.transpose(1, 0, 2, 3).reshape(DIM, NPIX).astype(jnp.bfloat16)
    zsq = jnp.sum(ze ** 2, axis=1).reshape(1, NPIX)
    esq = jnp.sum(codebook ** 2, axis=1).reshape(K, 1)
    emb2 = codebook.astype(jnp.bfloat16) * jnp.bfloat16(-2.0)
    idx = pl.pallas_call(
        _vq_body,
        grid=(K // KCH,),
        in_specs=[
            pl.BlockSpec((KCH, 1), lambda i: (i, 0)),
            pl.BlockSpec((KCH, DIM), lambda i: (i, 0)),
            pl.BlockSpec((DIM, NPIX), lambda i: (0, 0)),
            pl.BlockSpec((1, NPIX), lambda i: (0, 0)),
        ],
        out_specs=pl.BlockSpec((1, NPIX), lambda i: (0, 0)),
        out_shape=jax.ShapeDtypeStruct((1, NPIX), jnp.int32),
        scratch_shapes=[
            pltpu.VMEM((1, NPIX), jnp.float32),
            pltpu.VMEM((1, NPIX), jnp.int32),
        ],
    )(esq, emb2, zb, zsq)
    return idx.reshape(NPIX)


HW16 = 16
HROWS = 8448
SPW = HROWS // NS
GCH = BPW // 2


def _sc_body(codebook_hbm, idxg_hbm, idxs_hbm, ones_hbm, zeros_hbm, zq_hbm, upart_hbm,
             idxg_v, idxs_v, rows_v, ones_v, hist_sp, sem):
    c = lax.axis_index("c")
    s = lax.axis_index("s")
    wid = s * NC + c
    pltpu.sync_copy(zeros_hbm.at[pl.ds(s * SPW, SPW)], hist_sp.at[pl.ds(s * SPW, SPW)])
    pltpu.sync_copy(idxg_hbm.at[wid], idxg_v)
    pltpu.sync_copy(idxs_hbm.at[wid], idxs_v)
    pltpu.sync_copy(ones_hbm, ones_v)
    pltpu.async_copy(codebook_hbm.at[idxg_v.at[0]], rows_v.at[pl.ds(0, GCH)], sem).wait()
    pltpu.async_copy(codebook_hbm.at[idxg_v.at[1]], rows_v.at[pl.ds(GCH, GCH)], sem).wait()
    pltpu.sync_copy(rows_v, zq_hbm.at[pl.ds(wid * BPW, BPW)])
    plsc.subcore_barrier()
    pltpu.sync_copy(ones_v, hist_sp.at[idxs_v.at[0]], add=True)
    pltpu.sync_copy(ones_v, hist_sp.at[idxs_v.at[1]], add=True)
    plsc.subcore_barrier()
    pltpu.sync_copy(hist_sp.at[pl.ds(s * SPW, SPW)], upart_hbm.at[c, pl.ds(s * SPW, SPW)])


@functools.cache
def _sc_gather_usage_fn():
    return functools.partial(
        pl.kernel,
        mesh=plsc.VectorSubcoreMesh(core_axis_name="c", subcore_axis_name="s"),
        compiler_params=pltpu.CompilerParams(use_tc_tiling_on_sc=False),
        out_type=[
            jax.ShapeDtypeStruct((NPAD, DIM), jnp.float32),
            jax.ShapeDtypeStruct((NC, HROWS, HW16), jnp.float32),
        ],
        scratch_types=[
            pltpu.VMEM((2, GCH), jnp.int32),
            pltpu.VMEM((2, GCH), jnp.int32),
            pltpu.VMEM((BPW, DIM), jnp.float32),
            pltpu.VMEM((GCH, HW16), jnp.float32),
            pltpu.VMEM_SHARED((HROWS, HW16), jnp.float32),
            pltpu.SemaphoreType.DMA,
        ],
    )(_sc_body)


def _sc_gather_usage(codebook, idxg, idxs, ones_in, zeros_in):
    return _sc_gather_usage_fn()(codebook, idxg, idxs, ones_in, zeros_in)


def _rolls9(hb, wp):
    cs = []
    for dy in (-1, 0, 1):
        for dx in (-1, 0, 1):
            s = dy * wp + dx
            cs.append(jnp.roll(hb, -s, axis=0) if s else hb)
    return jnp.concatenate(cs, axis=1)


def _mask(hp, wp, h, w):
    r = jax.lax.broadcasted_iota(jnp.int32, (hp * wp, 1), 0)
    y = r // wp
    x = r % wp
    ok = (y >= 1) & (y <= h) & (x >= 1) & (x <= w)
    return ok.astype(jnp.float32)


def _pad_ring(t, h, w, c):
    zx = jnp.zeros((h, 1, c), t.dtype)
    t = jnp.concatenate([zx, t, zx], axis=1)
    zy = jnp.zeros((1, w + 2, c), t.dtype)
    t = jnp.concatenate([zy, t, zy], axis=0)
    return t.reshape((h + 2) * (w + 2), c)


def _dec_body(zq_ref, wd1_ref, bd1_ref, w3a_ref, b3a_ref, w3b_ref, b3b_ref,
              w4a_ref, b4a_ref, w4b_ref, b4b_ref, wp2_ref, b2_ref,
              out_ref):
    f32 = jnp.float32
    m58 = _mask(58, 58, 56, 56)

    def conv3(h, w_ref, b_ref, wp):
        x9 = _rolls9(h.astype(jnp.bfloat16), wp)
        y = jnp.dot(x9, w_ref[...], preferred_element_type=f32)
        return (y + b_ref[...])

    zq = zq_ref[...].reshape(3136, DIM)
    h = _pad_ring(zq.reshape(56, 56, DIM), 56, 56, DIM)
    h = conv3(h, wd1_ref, bd1_ref, 58) * m58
    for wa, ba, wb, bb in ((w3a_ref, b3a_ref, w3b_ref, b3b_ref),
                           (w4a_ref, b4a_ref, w4b_ref, b4b_ref)):
        t = jnp.maximum(h, 0.0)
        t = conv3(t, wa, ba, 58) * m58
        t = jnp.maximum(t, 0.0)
        t = jnp.dot(t.astype(jnp.bfloat16), wb[...], preferred_element_type=f32) + bb[...]
        h = h + t * m58
    x9 = _rolls9(h.astype(jnp.bfloat16), 58)
    phases = []
    for p in (0, 1):
        prow = []
        for q in (0, 1):
            ph = jnp.dot(x9, wp2_ref[...][2 * p + q], preferred_element_type=f32) + b2_ref[...]
            ph = ph.reshape(58, 58, DIM)[1:57, 1:57, :]
            prow.append(ph.reshape(56, 1, 56, 1, DIM))
        phases.append(jnp.concatenate(prow, axis=3))
    up = jnp.concatenate(phases, axis=1).reshape(112 * 112, DIM)
    up = jnp.maximum(up, 0.0)
    out_ref[...] = up[None]


def _dec_body2(up_ref, wp3_ref, b3_ref, out_ref):
    f32 = jnp.float32
    h2 = _pad_ring(up_ref[...].reshape(112, 112, DIM), 112, 112, DIM)
    x9b = _rolls9(h2.astype(jnp.bfloat16), 114)
    xall = jnp.dot(x9b, wp3_ref[...], preferred_element_type=f32) + b3_ref[...]
    out_ref[...] = xall[None]


def _tap_w(w, ky, kx):
    return w[:, :, ky, kx].transpose(1, 0)


def _phase_w(w2, p, q, cout):
    cin = w2.shape[1]
    blocks = []
    for dy in (-1, 0, 1):
        for dx in (-1, 0, 1):
            ty = 2 * dy + 2 - p
            tx = 2 * dx + 2 - q
            if 0 <= ty <= 3 and 0 <= tx <= 3:
                blocks.append(_tap_w(w2, ty, tx))
            else:
                blocks.append(jnp.zeros((cin, cout), jnp.float32))
    return jnp.concatenate(blocks, axis=0)


def _cat9(w):
    return jnp.concatenate([_tap_w(w, ky, kx) for ky in range(3) for kx in range(3)], axis=0)


def _decoder(zq_flat, dec_w1, dec_b1, r3_w1, r3_b1, r3_w2, r3_b2, r4_w1, r4_b1, r4_w2, r4_b2, dect_w2, dect_b2, dect_w3, dect_b3):
    bf = jnp.bfloat16
    wd1 = _cat9(dec_w1).astype(bf)
    w3a = _cat9(r3_w1).astype(bf)
    w3b = _tap_w(r3_w2, 0, 0).astype(bf)
    w4a = _cat9(r4_w1).astype(bf)
    w4b = _tap_w(r4_w2, 0, 0).astype(bf)
    w2f = jnp.flip(dect_w2, (2, 3)).transpose(1, 0, 2, 3)
    wp2 = jnp.stack([_phase_w(w2f, p, q, DIM) for p in (0, 1) for q in (0, 1)]).astype(bf)
    w3f = jnp.flip(dect_w3, (2, 3)).transpose(1, 0, 2, 3)
    wp3p = jnp.concatenate([jnp.pad(_phase_w(w3f, p, q, 3), ((0, 0), (0, 5))) for p in (0, 1) for q in (0, 1)], axis=1).astype(bf)
    b2d = dec_b1.reshape(1, DIM)
    b3a = r3_b1.reshape(1, DIM)
    b3b = r3_b2.reshape(1, DIM)
    b4a = r4_b1.reshape(1, DIM)
    b4b = r4_b2.reshape(1, DIM)
    bt2 = dect_b2.reshape(1, DIM)
    bt3 = jnp.tile(jnp.pad(dect_b3, (0, 5)), 4).reshape(1, 32)

    full = lambda *shape: pl.BlockSpec(shape, lambda n: (0,) * len(shape))
    up = pl.pallas_call(
        _dec_body,
        grid=(2,),
        in_specs=[
            pl.BlockSpec((1, 3136, DIM), lambda n: (n, 0, 0)),
            full(288, DIM), full(1, DIM),
            full(288, DIM), full(1, DIM), full(DIM, DIM), full(1, DIM),
            full(288, DIM), full(1, DIM), full(DIM, DIM), full(1, DIM),
            full(4, 288, DIM), full(1, DIM),
        ],
        out_specs=pl.BlockSpec((1, 112 * 112, DIM), lambda n: (n, 0, 0)),
        out_shape=jax.ShapeDtypeStruct((2, 112 * 112, DIM), jnp.float32),
    )(zq_flat.reshape(2, 3136, DIM), wd1, b2d, w3a, b3a, w3b, b3b, w4a, b4a, w4b, b4b, wp2, bt2)
    xh = pl.pallas_call(
        _dec_body2,
        grid=(2,),
        in_specs=[
            pl.BlockSpec((1, 112 * 112, DIM), lambda n: (n, 0, 0)),
            full(288, 32), full(1, 32),
        ],
        out_specs=pl.BlockSpec((1, 114 * 114, 32), lambda n: (n, 0, 0)),
        out_shape=jax.ShapeDtypeStruct((2, 114 * 114, 32), jnp.float32),
    )(up, wp3p, bt3)
    xh = xh.reshape(2, 114, 114, 2, 2, 8)[:, 1:113, 1:113, :, :, :3]
    xh = xh.transpose(0, 5, 1, 3, 2, 4).reshape(2, 3, 224, 224)
    return xh


def kernel(x, enc_w1, enc_b1, enc_w2, enc_b2, enc_w3, enc_b3, r1_w1, r1_b1, r1_w2, r1_b2, r2_w1, r2_b1, r2_w2, r2_b2, codebook, dec_w1, dec_b1, r3_w1, r3_b1, r3_w2, r3_b2, r4_w1, r4_b1, r4_w2, r4_b2, dect_w2, dect_b2, dect_w3, dect_b3):
    h = _conv(x, enc_w1, enc_b1, 2, 1)
    h = jax.nn.relu(h)
    h = _conv(h, enc_w2, enc_b2, 2, 1)
    h = jax.nn.relu(h)
    h = _conv(h, enc_w3, enc_b3, 1, 1)
    h = _res(h, r1_w1, r1_b1, r1_w2, r1_b2)
    ze = _res(h, r2_w1, r2_b1, r2_w2, r2_b2)

    idx_flat = _vq_argmin(ze, codebook)
    idxg = jnp.concatenate([idx_flat, jnp.zeros((NPAD - NPIX,), jnp.int32)]).reshape(NW, 2, GCH)
    idxs = jnp.concatenate([idx_flat, jnp.full((NPAD - NPIX,), K, jnp.int32)]).reshape(NW, 2, GCH)
    ones_in = jnp.ones((GCH, HW16), jnp.float32)
    zeros_in = jnp.zeros((HROWS, HW16), jnp.float32)

    zq_flat, upart = _sc_gather_usage(codebook, idxg, idxs, ones_in, zeros_in)
    usage = (upart[0, :K, 0] + upart[1, :K, 0]) / jnp.float32(NPIX)
    zq = zq_flat[:NPIX].reshape(2, 56, 56, DIM).transpose(0, 3, 1, 2)

    x_hat = _decoder(zq_flat[:NPIX], dec_w1, dec_b1, r3_w1, r3_b1, r3_w2, r3_b2,
                     r4_w1, r4_b1, r4_w2, r4_b2, dect_w2, dect_b2, dect_w3, dect_b3)
    return (x_hat, ze, zq, usage)


# speedup vs baseline: 1.0976x; 1.0976x over previous
import functools
import jax, jax.numpy as jnp
from jax import lax
from jax.experimental import pallas as pl
from jax.experimental.pallas import tpu as pltpu
from jax.experimental.pallas import tpu_sc as plsc

K = 8192
DIM = 32
NPIX = 6272
KCH = 1024

NC = 2
NS = 16
NW = NC * NS
BPW = 208
NPAD = NW * BPW
KPW = K // NW


def _conv(x, w, b, s, p):
    y = jax.lax.conv_general_dilated(x, w, (s, s), [(p, p), (p, p)], dimension_numbers=('NCHW', 'OIHW', 'NCHW'))
    return y + b[None, :, None, None]


def _convt(x, w, b):
    w2 = jnp.flip(w, (2, 3)).transpose(1, 0, 2, 3)
    y = jax.lax.conv_general_dilated(x, w2, (1, 1), [(2, 2), (2, 2)], lhs_dilation=(2, 2), dimension_numbers=('NCHW', 'OIHW', 'NCHW'))
    return y + b[None, :, None, None]


def _res(x, w1, b1, w2, b2):
    t = jax.nn.relu(x)
    t = _conv(t, w1, b1, 1, 1)
    t = jax.nn.relu(t)
    t = _conv(t, w2, b2, 1, 0)
    return x + t


def _vq_body(esq_ref, emb2_ref, zb_ref, zsq_ref, idx_ref, best_ref, bidx_ref):
    i = pl.program_id(0)
    cross2 = jax.lax.dot_general(
        emb2_ref[...], zb_ref[...], (((1,), (0,)), ((), ())),
        preferred_element_type=jnp.float32)
    a = esq_ref[...] + cross2
    d = a + zsq_ref[...]
    cmin = jnp.min(d, axis=0, keepdims=True)
    rows = jax.lax.broadcasted_iota(jnp.int32, d.shape, 0)
    loc = jnp.min(jnp.where(d == cmin, rows, jnp.int32(2 ** 30)), axis=0, keepdims=True)
    gidx = loc + i * KCH

    @pl.when(i == 0)
    def _():
        best_ref[...] = cmin
        bidx_ref[...] = gidx

    @pl.when(i > 0)
    def _():
        upd = cmin < best_ref[...]
        bidx_ref[...] = jnp.where(upd, gidx, bidx_ref[...])
        best_ref[...] = jnp.where(upd, cmin, best_ref[...])

    @pl.when(i == pl.num_programs(0) - 1)
    def _():
        idx_ref[...] = bidx_ref[...]


def _vq_argmin(ze, codebook):
    zb = ze.transpose(1, 0, 2, 3).reshape(DIM, NPIX).astype(jnp.bfloat16)
    zsq = jnp.sum(ze ** 2, axis=1).reshape(1, NPIX)
    esq = jnp.sum(codebook ** 2, axis=1).reshape(K, 1)
    emb2 = codebook.astype(jnp.bfloat16) * jnp.bfloat16(-2.0)
    idx = pl.pallas_call(
        _vq_body,
        grid=(K // KCH,),
        in_specs=[
            pl.BlockSpec((KCH, 1), lambda i: (i, 0)),
            pl.BlockSpec((KCH, DIM), lambda i: (i, 0)),
            pl.BlockSpec((DIM, NPIX), lambda i: (0, 0)),
            pl.BlockSpec((1, NPIX), lambda i: (0, 0)),
        ],
        out_specs=pl.BlockSpec((1, NPIX), lambda i: (0, 0)),
        out_shape=jax.ShapeDtypeStruct((1, NPIX), jnp.int32),
        scratch_shapes=[
            pltpu.VMEM((1, NPIX), jnp.float32),
            pltpu.VMEM((1, NPIX), jnp.int32),
        ],
    )(esq, emb2, zb, zsq)
    return idx.reshape(NPIX)


HW8 = 8
HROWS = 8448
SPW = HROWS // NS
GCH = BPW // 2


def _sc_gather_body(codebook_hbm, idxg_hbm, zq_hbm, idxg_v, rows_v, sem):
    c = lax.axis_index("c")
    s = lax.axis_index("s")
    wid = s * NC + c
    pltpu.sync_copy(idxg_hbm.at[wid], idxg_v)
    pltpu.async_copy(codebook_hbm.at[idxg_v.at[0]], rows_v.at[pl.ds(0, GCH)], sem).wait()
    pltpu.async_copy(codebook_hbm.at[idxg_v.at[1]], rows_v.at[pl.ds(GCH, GCH)], sem).wait()
    pltpu.sync_copy(rows_v, zq_hbm.at[pl.ds(wid * BPW, BPW)])


def _sc_usage_body(idxs_hbm, ones_hbm, zeros_hbm, upart_hbm, idxs_v, ones_v, hist_sp, sem):
    c = lax.axis_index("c")
    s = lax.axis_index("s")
    wid = s * NC + c
    pltpu.sync_copy(zeros_hbm.at[pl.ds(s * SPW, SPW)], hist_sp.at[pl.ds(s * SPW, SPW)])
    pltpu.sync_copy(idxs_hbm.at[wid], idxs_v)
    pltpu.sync_copy(ones_hbm, ones_v)
    plsc.subcore_barrier()
    pltpu.sync_copy(ones_v, hist_sp.at[idxs_v.at[0]], add=True)
    pltpu.sync_copy(ones_v, hist_sp.at[idxs_v.at[1]], add=True)
    plsc.subcore_barrier()
    pltpu.sync_copy(hist_sp.at[pl.ds(s * SPW, SPW)], upart_hbm.at[c, pl.ds(s * SPW, SPW)])


@functools.cache
def _sc_fns():
    mesh = plsc.VectorSubcoreMesh(core_axis_name="c", subcore_axis_name="s")
    cp = pltpu.CompilerParams(use_tc_tiling_on_sc=False)
    gather = functools.partial(
        pl.kernel, mesh=mesh, compiler_params=cp,
        out_type=[jax.ShapeDtypeStruct((NPAD, DIM), jnp.float32)],
        scratch_types=[
            pltpu.VMEM((2, GCH), jnp.int32),
            pltpu.VMEM((BPW, DIM), jnp.float32),
            pltpu.SemaphoreType.DMA,
        ],
    )(_sc_gather_body)
    usage = functools.partial(
        pl.kernel, mesh=mesh, compiler_params=cp,
        out_type=[jax.ShapeDtypeStruct((NC, HROWS, HW8), jnp.float32)],
        scratch_types=[
            pltpu.VMEM((2, GCH), jnp.int32),
            pltpu.VMEM((GCH, HW8), jnp.float32),
            pltpu.VMEM_SHARED((HROWS, HW8), jnp.float32),
            pltpu.SemaphoreType.DMA,
        ],
    )(_sc_usage_body)
    return gather, usage


def _rolls9(hb, wp):
    cs = []
    for dy in (-1, 0, 1):
        for dx in (-1, 0, 1):
            s = dy * wp + dx
            cs.append(jnp.roll(hb, -s, axis=0) if s else hb)
    return jnp.concatenate(cs, axis=1)


def _mask(hp, wp, h, w):
    r = jax.lax.broadcasted_iota(jnp.int32, (hp * wp, 1), 0)
    y = r // wp
    x = r % wp
    ok = (y >= 1) & (y <= h) & (x >= 1) & (x <= w)
    return ok.astype(jnp.float32)


def _pad_ring(t, h, w, c):
    zx = jnp.zeros((h, 1, c), t.dtype)
    t = jnp.concatenate([zx, t, zx], axis=1)
    zy = jnp.zeros((1, w + 2, c), t.dtype)
    t = jnp.concatenate([zy, t, zy], axis=0)
    return t.reshape((h + 2) * (w + 2), c)


def _dec_body(zq_ref, wd1_ref, bd1_ref, w3a_ref, b3a_ref, w3b_ref, b3b_ref,
              w4a_ref, b4a_ref, w4b_ref, b4b_ref, wp2_ref, b2_ref,
              out_ref):
    f32 = jnp.float32
    m58 = _mask(58, 58, 56, 56)

    def conv3(h, w_ref, b_ref, wp):
        x9 = _rolls9(h.astype(jnp.bfloat16), wp)
        y = jnp.dot(x9, w_ref[...], preferred_element_type=f32)
        return (y + b_ref[...])

    zq = zq_ref[...].reshape(3136, DIM)
    h = _pad_ring(zq.reshape(56, 56, DIM), 56, 56, DIM)
    h = conv3(h, wd1_ref, bd1_ref, 58) * m58
    for wa, ba, wb, bb in ((w3a_ref, b3a_ref, w3b_ref, b3b_ref),
                           (w4a_ref, b4a_ref, w4b_ref, b4b_ref)):
        t = jnp.maximum(h, 0.0)
        t = conv3(t, wa, ba, 58) * m58
        t = jnp.maximum(t, 0.0)
        t = jnp.dot(t.astype(jnp.bfloat16), wb[...], preferred_element_type=f32) + bb[...]
        h = h + t * m58
    x9 = _rolls9(h.astype(jnp.bfloat16), 58)
    phases = []
    for p in (0, 1):
        prow = []
        for q in (0, 1):
            ph = jnp.dot(x9, wp2_ref[...][2 * p + q], preferred_element_type=f32) + b2_ref[...]
            ph = ph.reshape(58, 58, DIM)[1:57, 1:57, :]
            prow.append(ph.reshape(56, 1, 56, 1, DIM))
        phases.append(jnp.concatenate(prow, axis=3))
    up = jnp.concatenate(phases, axis=1).reshape(112 * 112, DIM)
    up = jnp.maximum(up, 0.0)
    out_ref[...] = up[None]


def _dec_body2(up_ref, wp3_ref, b3_ref, out_ref):
    f32 = jnp.float32
    h2 = _pad_ring(up_ref[...].reshape(112, 112, DIM), 112, 112, DIM)
    x9b = _rolls9(h2.astype(jnp.bfloat16), 114)
    xall = jnp.dot(x9b, wp3_ref[...], preferred_element_type=f32) + b3_ref[...]
    out_ref[...] = xall[None]


def _tap_w(w, ky, kx):
    return w[:, :, ky, kx].transpose(1, 0)


def _phase_w(w2, p, q, cout):
    cin = w2.shape[1]
    blocks = []
    for dy in (-1, 0, 1):
        for dx in (-1, 0, 1):
            ty = 2 * dy + 2 - p
            tx = 2 * dx + 2 - q
            if 0 <= ty <= 3 and 0 <= tx <= 3:
                blocks.append(_tap_w(w2, ty, tx))
            else:
                blocks.append(jnp.zeros((cin, cout), jnp.float32))
    return jnp.concatenate(blocks, axis=0)


def _cat9(w):
    return jnp.concatenate([_tap_w(w, ky, kx) for ky in range(3) for kx in range(3)], axis=0)


def _decoder(zq_flat, dec_w1, dec_b1, r3_w1, r3_b1, r3_w2, r3_b2, r4_w1, r4_b1, r4_w2, r4_b2, dect_w2, dect_b2, dect_w3, dect_b3):
    bf = jnp.bfloat16
    wd1 = _cat9(dec_w1).astype(bf)
    w3a = _cat9(r3_w1).astype(bf)
    w3b = _tap_w(r3_w2, 0, 0).astype(bf)
    w4a = _cat9(r4_w1).astype(bf)
    w4b = _tap_w(r4_w2, 0, 0).astype(bf)
    w2f = jnp.flip(dect_w2, (2, 3)).transpose(1, 0, 2, 3)
    wp2 = jnp.stack([_phase_w(w2f, p, q, DIM) for p in (0, 1) for q in (0, 1)]).astype(bf)
    w3f = jnp.flip(dect_w3, (2, 3)).transpose(1, 0, 2, 3)
    wp3p = jnp.concatenate([jnp.pad(_phase_w(w3f, p, q, 3), ((0, 0), (0, 5))) for p in (0, 1) for q in (0, 1)], axis=1).astype(bf)
    b2d = dec_b1.reshape(1, DIM)
    b3a = r3_b1.reshape(1, DIM)
    b3b = r3_b2.reshape(1, DIM)
    b4a = r4_b1.reshape(1, DIM)
    b4b = r4_b2.reshape(1, DIM)
    bt2 = dect_b2.reshape(1, DIM)
    bt3 = jnp.tile(jnp.pad(dect_b3, (0, 5)), 4).reshape(1, 32)

    full = lambda *shape: pl.BlockSpec(shape, lambda n: (0,) * len(shape))
    up = pl.pallas_call(
        _dec_body,
        grid=(2,),
        in_specs=[
            pl.BlockSpec((1, 3136, DIM), lambda n: (n, 0, 0)),
            full(288, DIM), full(1, DIM),
            full(288, DIM), full(1, DIM), full(DIM, DIM), full(1, DIM),
            full(288, DIM), full(1, DIM), full(DIM, DIM), full(1, DIM),
            full(4, 288, DIM), full(1, DIM),
        ],
        out_specs=pl.BlockSpec((1, 112 * 112, DIM), lambda n: (n, 0, 0)),
        out_shape=jax.ShapeDtypeStruct((2, 112 * 112, DIM), jnp.float32),
    )(zq_flat.reshape(2, 3136, DIM), wd1, b2d, w3a, b3a, w3b, b3b, w4a, b4a, w4b, b4b, wp2, bt2)
    xh = pl.pallas_call(
        _dec_body2,
        grid=(2,),
        in_specs=[
            pl.BlockSpec((1, 112 * 112, DIM), lambda n: (n, 0, 0)),
            full(288, 32), full(1, 32),
        ],
        out_specs=pl.BlockSpec((1, 114 * 114, 32), lambda n: (n, 0, 0)),
        out_shape=jax.ShapeDtypeStruct((2, 114 * 114, 32), jnp.float32),
    )(up, wp3p, bt3)
    xh = xh.reshape(2, 114, 114, 2, 2, 8)[:, 1:113, 1:113, :, :, :3]
    xh = xh.transpose(0, 5, 1, 3, 2, 4).reshape(2, 3, 224, 224)
    return xh


def kernel(x, enc_w1, enc_b1, enc_w2, enc_b2, enc_w3, enc_b3, r1_w1, r1_b1, r1_w2, r1_b2, r2_w1, r2_b1, r2_w2, r2_b2, codebook, dec_w1, dec_b1, r3_w1, r3_b1, r3_w2, r3_b2, r4_w1, r4_b1, r4_w2, r4_b2, dect_w2, dect_b2, dect_w3, dect_b3):
    h = _conv(x, enc_w1, enc_b1, 2, 1)
    h = jax.nn.relu(h)
    h = _conv(h, enc_w2, enc_b2, 2, 1)
    h = jax.nn.relu(h)
    h = _conv(h, enc_w3, enc_b3, 1, 1)
    h = _res(h, r1_w1, r1_b1, r1_w2, r1_b2)
    ze = _res(h, r2_w1, r2_b1, r2_w2, r2_b2)

    idx_flat = _vq_argmin(ze, codebook)
    idxg = jnp.concatenate([idx_flat, jnp.zeros((NPAD - NPIX,), jnp.int32)]).reshape(NW, 2, GCH)
    idxs = jnp.concatenate([idx_flat, jnp.full((NPAD - NPIX,), K, jnp.int32)]).reshape(NW, 2, GCH)
    ones_in = jnp.ones((GCH, HW8), jnp.float32)
    zeros_in = jnp.zeros((HROWS, HW8), jnp.float32)

    sc_gather, sc_usage = _sc_fns()
    (zq_flat,) = sc_gather(codebook, idxg)
    (upart,) = sc_usage(idxs, ones_in, zeros_in)
    usage = (upart[0, :K, 0] + upart[1, :K, 0]) / jnp.float32(NPIX)
    zq = zq_flat[:NPIX].reshape(2, 56, 56, DIM).transpose(0, 3, 1, 2)

    h = _conv(zq, dec_w1, dec_b1, 1, 1)
    h = _res(h, r3_w1, r3_b1, r3_w2, r3_b2)
    h = _res(h, r4_w1, r4_b1, r4_w2, r4_b2)
    h = _convt(h, dect_w2, dect_b2)
    h = jax.nn.relu(h)
    x_hat = _convt(h, dect_w3, dect_b3)
    return (x_hat, ze, zq, usage)
